# baseline (speedup 1.0000x reference)
import jax
import jax.numpy as jnp
from jax import lax
from jax.experimental import pallas as pl
from jax.experimental.pallas import tpu as pltpu
from jax.experimental.pallas import tpu_sc as plsc

N_NODES = 10000
D = 256
DH = 128
E = 160000
NC = 2
NS = 16
NP = 2
EPT = E // NS
CH = 32
CAP = EPT + CH
NH = 5120
TRASH = NH
ACC_R = 5248
SG = 2000
NSG = EPT // SG
ZCH = (64, 64, 64, 64, 64, 8)
ECH = (64, 64, 64, 64, 64)


def _matmul_body(x_ref, w_ref, o_ref):
    o_ref[0] = jnp.dot(x_ref[...], w_ref[0], preferred_element_type=jnp.float32)


def _matmul(x, W2):
    nb = 10
    bn = N_NODES // nb
    return pl.pallas_call(
        _matmul_body,
        grid=(NC, nb),
        in_specs=[
            pl.BlockSpec((bn, D), lambda c, i: (i, 0)),
            pl.BlockSpec((1, D, DH), lambda c, i: (c, 0, 0)),
        ],
        out_specs=pl.BlockSpec((1, bn, DH), lambda c, i: (c, i, 0)),
        out_shape=jax.ShapeDtypeStruct((NC, N_NODES, DH), jnp.float32),
    )(x, W2)


def _sc_body(h_hbm, src_hbm, dst_hbm, w_hbm, b_hbm, a_hbm, out_hbm,
             sbs, sbd, sbw, srcp, dstp, wp, dstc, rows0, rows1, obuf,
             bvec, avec, acc, semg0, semg1, sems0, sems1):
    c = lax.axis_index("c")
    t = lax.axis_index("s")
    col0 = c * DH

    pltpu.sync_copy(b_hbm.at[pl.ds(col0, DH)], bvec)
    pltpu.sync_copy(a_hbm.at[pl.ds(col0, DH)], avec)

    zeros16 = jnp.zeros((16,), jnp.float32)
    trash16 = jnp.full((16,), TRASH, jnp.int32)
    hoff = (c * N_NODES).astype(jnp.int32)

    def _fill(i, _):
        sl = pl.ds(i * 16, 16)
        srcp[sl] = jnp.zeros((16,), jnp.int32)
        dstp[sl] = trash16
        wp[sl] = zeros16
        return 0
    lax.fori_loop(0, CAP // 16, _fill, 0)

    def _part(i, carry):
        n0, m = carry
        sl = pl.ds(i * 16, 16)
        s = sbs[0, sl] + hoff
        d = sbd[0, sl]
        w = sbw[0, sl]
        m1 = d >= NH
        m0 = jnp.logical_not(m1)
        cnt1 = jnp.sum(m1.astype(jnp.int32))
        plsc.store_compressed(srcp.at[pl.ds(n0, 16)], s, mask=m0)
        plsc.store_compressed(dstp.at[pl.ds(n0, 16)], d, mask=m0)
        plsc.store_compressed(wp.at[pl.ds(n0, 16)], w, mask=m0)
        m_new = m - cnt1
        plsc.store_compressed(srcp.at[pl.ds(m_new, 16)], s, mask=m1)
        plsc.store_compressed(dstp.at[pl.ds(m_new, 16)], d - NH, mask=m1)
        plsc.store_compressed(wp.at[pl.ds(m_new, 16)], w, mask=m1)
        return (n0 + (16 - cnt1), m_new)
    carry = (jnp.int32(0), jnp.int32(CAP))
    for g in range(NSG):
        pltpu.sync_copy(src_hbm.at[t, g], sbs)
        pltpu.sync_copy(dst_hbm.at[t, g], sbd)
        pltpu.sync_copy(w_hbm.at[t, g], sbw)
        carry = lax.fori_loop(0, SG // 16, _part, carry)
    n0, m = carry
    trips0 = (n0 + (CH - 1)) // CH
    trips1 = ((CAP - m) + (CH - 1)) // CH

    rowsb = (rows0, rows1)
    semg = (semg0, semg1)
    sems = (sems0, sems1)

    def _issue_gather(b, off):
        pltpu.async_copy(h_hbm.at[srcp.at[pl.ds(off, CH)]], rowsb[b], semg[b])

    def _wait_gather(b, off):
        pltpu.make_async_copy(
            h_hbm.at[srcp.at[pl.ds(off, CH)]], rowsb[b], semg[b]).wait()

    def _issue_scatter(b):
        pltpu.async_copy(rowsb[b], acc.at[dstc.at[b]], sems[b], add=True)

    def _drain_scatter(b):
        pltpu.make_async_copy(rowsb[b], acc.at[dstc.at[b]], sems[b]).wait()

    def _scale(b, off):
        r = rowsb[b]

        def _edge(i, _):
            for u in range(4):
                e = i * 4 + u
                wb = plsc.load_gather(wp, [jnp.full((16,), off + e, jnp.int32)])
                for v in range(DH // 16):
                    sl = pl.ds(v * 16, 16)
                    r[e, sl] = r[e, sl] * wb
            return 0
        lax.fori_loop(0, CH // 4, _edge, 0)
        for k in range(CH // 16):
            dstc[b, pl.ds(k * 16, 16)] = dstp[pl.ds(off + k * 16, 16)]

    def _bucket_loop(p, trips):
        def off_of(jj):
            if p == 0:
                return jj * CH
            return CAP - CH * (jj + 1)

        @pl.when(trips > 0)
        def _():
            _issue_gather(0, off_of(0))

        def _body(i, _):
            for b in range(2):
                jj = 2 * i + b

                @pl.when(jj < trips)
                def _(jj=jj, b=b):
                    _wait_gather(b, off_of(jj))

                    @pl.when(jj + 1 < trips)
                    def _(jj=jj, b=b):
                        @pl.when(jj >= 1)
                        def _(b=b):
                            _drain_scatter(b ^ 1)
                        _issue_gather(b ^ 1, off_of(jj + 1))

                    _scale(b, off_of(jj))
                    _issue_scatter(b)
            return 0
        lax.fori_loop(0, (trips + 1) // 2, _body, 0)

        @pl.when(trips >= 1)
        def _():
            _drain_scatter(0)

        @pl.when(trips >= 2)
        def _():
            _drain_scatter(1)

    for p in range(NP):
        lo = p * NH

        def _zero(r, _):
            for v in range(DH // 16):
                obuf[r, pl.ds(v * 16, 16)] = zeros16
            return 0
        lax.fori_loop(0, max(ZCH), _zero, 0)
        r0 = t * (ACC_R // NS)
        for sz in ZCH:
            pltpu.sync_copy(obuf.at[pl.ds(0, sz)], acc.at[pl.ds(r0, sz)])
            r0 += sz
        plsc.subcore_barrier()

        _bucket_loop(p, trips0 if p == 0 else trips1)
        plsc.subcore_barrier()

        r0 = t * (NH // NS)
        for sz in ECH:
            g0 = lo + r0

            @pl.when(g0 < N_NODES)
            def _(r0=r0, g0=g0):
                pltpu.sync_copy(acc.at[pl.ds(r0, sz)], obuf.at[pl.ds(0, sz)])

                def _act(r, _):
                    for v in range(DH // 16):
                        sl = pl.ds(v * 16, 16)
                        tv = obuf[r, sl] + bvec[sl]
                        obuf[r, sl] = jnp.where(tv >= 0.0, tv, avec[sl] * tv)
                    return 0
                lax.fori_loop(0, sz, _act, 0)

                @pl.when(g0 + sz <= N_NODES)
                def _():
                    pltpu.sync_copy(
                        obuf.at[pl.ds(0, sz)],
                        out_hbm.at[pl.ds(g0, sz), pl.ds(col0, DH)])

                @pl.when(g0 + sz > N_NODES)
                def _():
                    part = N_NODES % sz
                    pltpu.sync_copy(
                        obuf.at[pl.ds(0, part)],
                        out_hbm.at[pl.ds(g0, part), pl.ds(col0, DH)])
            r0 += sz
        plsc.subcore_barrier()


def _sc_aggregate(h2f, src_r, dst_r, w_r, b, alpha):
    mesh = plsc.VectorSubcoreMesh(
        core_axis_name="c", subcore_axis_name="s", num_cores=NC, num_subcores=NS)
    f = pl.kernel(
        _sc_body,
        out_type=jax.ShapeDtypeStruct((N_NODES, D), jnp.float32),
        mesh=mesh,
        compiler_params=pltpu.CompilerParams(needs_layout_passes=False),
        scratch_types=[
            pltpu.VMEM((1, SG), jnp.int32),
            pltpu.VMEM((1, SG), jnp.int32),
            pltpu.VMEM((1, SG), jnp.float32),
            pltpu.VMEM((CAP,), jnp.int32),
            pltpu.VMEM((CAP,), jnp.int32),
            pltpu.VMEM((CAP,), jnp.float32),
            pltpu.VMEM((2, CH), jnp.int32),
            pltpu.VMEM((CH, DH), jnp.float32),
            pltpu.VMEM((CH, DH), jnp.float32),
            pltpu.VMEM((max(ZCH), DH), jnp.float32),
            pltpu.VMEM((DH,), jnp.float32),
            pltpu.VMEM((DH,), jnp.float32),
            pltpu.VMEM_SHARED((ACC_R, DH), jnp.float32),
            pltpu.SemaphoreType.DMA,
            pltpu.SemaphoreType.DMA,
            pltpu.SemaphoreType.DMA,
            pltpu.SemaphoreType.DMA,
        ],
    )
    return f(h2f, src_r, dst_r, w_r, b, alpha)


@jax.jit
def kernel(x, edge_index, weights, W, b, alpha):
    src = edge_index[0].astype(jnp.int32).reshape(NS, NSG, 1, SG)
    dst = edge_index[1].astype(jnp.int32).reshape(NS, NSG, 1, SG)
    w_r = weights.reshape(NS, NSG, 1, SG)
    W2 = W.reshape(D, NC, DH).transpose(1, 0, 2)
    h2 = _matmul(x, W2)
    h2f = h2.reshape(NC * N_NODES, DH)
    return _sc_aggregate(h2f, src, dst, w_r, b, alpha)


# speedup vs baseline: 1.1220x; 1.1220x over previous
import jax
import jax.numpy as jnp
from jax import lax
from jax.experimental import pallas as pl
from jax.experimental.pallas import tpu as pltpu
from jax.experimental.pallas import tpu_sc as plsc

N_NODES = 10000
D = 256
DH = 128
E = 160000
NC = 2
NS = 16
NP = 2
EPT = E // NS
CH = 48
CAP = EPT + CH
NH = 5120
TRASH = NH
ACC_R = 5248
SG = 2000
NSG = EPT // SG
ZCH = (64, 64, 64, 64, 64, 8)
ECH = (64, 64, 64, 64, 64)


def _matmul_body(x_ref, w_ref, o_ref):
    o_ref[0] = jnp.dot(x_ref[...], w_ref[0], preferred_element_type=jnp.float32)


def _matmul(x, W2):
    nb = 10
    bn = N_NODES // nb
    return pl.pallas_call(
        _matmul_body,
        grid=(NC, nb),
        in_specs=[
            pl.BlockSpec((bn, D), lambda c, i: (i, 0)),
            pl.BlockSpec((1, D, DH), lambda c, i: (c, 0, 0)),
        ],
        out_specs=pl.BlockSpec((1, bn, DH), lambda c, i: (c, i, 0)),
        out_shape=jax.ShapeDtypeStruct((NC, N_NODES, DH), jnp.float32),
    )(x, W2)


def _sc_body(h_hbm, src_hbm, dst_hbm, w_hbm, b_hbm, a_hbm, out_hbm,
             sbs, sbd, sbw, srcp, dstp, wp, dstc, rows0, rows1, obuf,
             bvec, avec, acc, semg0, semg1, sems0, sems1):
    c = lax.axis_index("c")
    t = lax.axis_index("s")
    col0 = c * DH

    pltpu.sync_copy(b_hbm.at[pl.ds(col0, DH)], bvec)
    pltpu.sync_copy(a_hbm.at[pl.ds(col0, DH)], avec)

    zeros16 = jnp.zeros((16,), jnp.float32)
    trash16 = jnp.full((16,), TRASH, jnp.int32)
    hoff = (c * N_NODES).astype(jnp.int32)

    def _fill(i, _):
        sl = pl.ds(i * 16, 16)
        srcp[sl] = jnp.zeros((16,), jnp.int32)
        dstp[sl] = trash16
        wp[sl] = zeros16
        return 0
    lax.fori_loop(0, CAP // 16, _fill, 0)

    def _part(i, carry):
        n0, m = carry
        sl = pl.ds(i * 16, 16)
        s = sbs[0, sl] + hoff
        d = sbd[0, sl]
        w = sbw[0, sl]
        m1 = d >= NH
        m0 = jnp.logical_not(m1)
        cnt1 = jnp.sum(m1.astype(jnp.int32))
        plsc.store_compressed(srcp.at[pl.ds(n0, 16)], s, mask=m0)
        plsc.store_compressed(dstp.at[pl.ds(n0, 16)], d, mask=m0)
        plsc.store_compressed(wp.at[pl.ds(n0, 16)], w, mask=m0)
        m_new = m - cnt1
        plsc.store_compressed(srcp.at[pl.ds(m_new, 16)], s, mask=m1)
        plsc.store_compressed(dstp.at[pl.ds(m_new, 16)], d - NH, mask=m1)
        plsc.store_compressed(wp.at[pl.ds(m_new, 16)], w, mask=m1)
        return (n0 + (16 - cnt1), m_new)
    carry = (jnp.int32(0), jnp.int32(CAP))
    for g in range(NSG):
        pltpu.sync_copy(src_hbm.at[t, g], sbs)
        pltpu.sync_copy(dst_hbm.at[t, g], sbd)
        pltpu.sync_copy(w_hbm.at[t, g], sbw)
        carry = lax.fori_loop(0, SG // 16, _part, carry)
    n0, m = carry
    trips0 = (n0 + (CH - 1)) // CH
    trips1 = ((CAP - m) + (CH - 1)) // CH

    rowsb = (rows0, rows1)
    semg = (semg0, semg1)
    sems = (sems0, sems1)

    def _issue_gather(b, off):
        pltpu.async_copy(h_hbm.at[srcp.at[pl.ds(off, CH)]], rowsb[b], semg[b])

    def _wait_gather(b, off):
        pltpu.make_async_copy(
            h_hbm.at[srcp.at[pl.ds(off, CH)]], rowsb[b], semg[b]).wait()

    def _issue_scatter(b):
        pltpu.async_copy(rowsb[b], acc.at[dstc.at[b]], sems[b], add=True)

    def _drain_scatter(b):
        pltpu.make_async_copy(rowsb[b], acc.at[dstc.at[b]], sems[b]).wait()

    def _scale(b, off):
        r = rowsb[b]

        def _edge(i, _):
            for u in range(4):
                e = i * 4 + u
                wb = plsc.load_gather(wp, [jnp.full((16,), off + e, jnp.int32)])
                for v in range(DH // 16):
                    sl = pl.ds(v * 16, 16)
                    r[e, sl] = r[e, sl] * wb
            return 0
        lax.fori_loop(0, CH // 4, _edge, 0)
        for k in range(CH // 16):
            dstc[b, pl.ds(k * 16, 16)] = dstp[pl.ds(off + k * 16, 16)]

    def _bucket_loop(p, trips):
        def off_of(jj):
            if p == 0:
                return jj * CH
            return CAP - CH * (jj + 1)

        @pl.when(trips > 0)
        def _():
            _issue_gather(0, off_of(0))

        def _body(i, _):
            for b in range(2):
                jj = 2 * i + b

                @pl.when(jj < trips)
                def _(jj=jj, b=b):
                    _wait_gather(b, off_of(jj))

                    @pl.when(jj + 1 < trips)
                    def _(jj=jj, b=b):
                        @pl.when(jj >= 1)
                        def _(b=b):
                            _drain_scatter(b ^ 1)
                        _issue_gather(b ^ 1, off_of(jj + 1))

                    _scale(b, off_of(jj))
                    _issue_scatter(b)
            return 0
        lax.fori_loop(0, (trips + 1) // 2, _body, 0)

        @pl.when(trips >= 1)
        def _():
            _drain_scatter(0)

        @pl.when(trips >= 2)
        def _():
            _drain_scatter(1)

    for p in range(NP):
        lo = p * NH

        def _zero(r, _):
            for v in range(DH // 16):
                obuf[r, pl.ds(v * 16, 16)] = zeros16
            return 0
        lax.fori_loop(0, max(ZCH), _zero, 0)
        r0 = t * (ACC_R // NS)
        for sz in ZCH:
            pltpu.sync_copy(obuf.at[pl.ds(0, sz)], acc.at[pl.ds(r0, sz)])
            r0 += sz
        plsc.subcore_barrier()

        _bucket_loop(p, trips0 if p == 0 else trips1)
        plsc.subcore_barrier()

        r0 = t * (NH // NS)
        for sz in ECH:
            g0 = lo + r0

            @pl.when(g0 < N_NODES)
            def _(r0=r0, g0=g0):
                pltpu.sync_copy(acc.at[pl.ds(r0, sz)], obuf.at[pl.ds(0, sz)])

                def _act(r, _):
                    for v in range(DH // 16):
                        sl = pl.ds(v * 16, 16)
                        tv = obuf[r, sl] + bvec[sl]
                        obuf[r, sl] = jnp.where(tv >= 0.0, tv, avec[sl] * tv)
                    return 0
                lax.fori_loop(0, sz, _act, 0)

                @pl.when(g0 + sz <= N_NODES)
                def _():
                    pltpu.sync_copy(
                        obuf.at[pl.ds(0, sz)],
                        out_hbm.at[pl.ds(g0, sz), pl.ds(col0, DH)])

                @pl.when(g0 + sz > N_NODES)
                def _():
                    part = N_NODES % sz
                    pltpu.sync_copy(
                        obuf.at[pl.ds(0, part)],
                        out_hbm.at[pl.ds(g0, part), pl.ds(col0, DH)])
            r0 += sz
        plsc.subcore_barrier()


def _sc_aggregate(h2f, src_r, dst_r, w_r, b, alpha):
    mesh = plsc.VectorSubcoreMesh(
        core_axis_name="c", subcore_axis_name="s", num_cores=NC, num_subcores=NS)
    f = pl.kernel(
        _sc_body,
        out_type=jax.ShapeDtypeStruct((N_NODES, D), jnp.float32),
        mesh=mesh,
        compiler_params=pltpu.CompilerParams(needs_layout_passes=False),
        scratch_types=[
            pltpu.VMEM((1, SG), jnp.int32),
            pltpu.VMEM((1, SG), jnp.int32),
            pltpu.VMEM((1, SG), jnp.float32),
            pltpu.VMEM((CAP,), jnp.int32),
            pltpu.VMEM((CAP,), jnp.int32),
            pltpu.VMEM((CAP,), jnp.float32),
            pltpu.VMEM((2, CH), jnp.int32),
            pltpu.VMEM((CH, DH), jnp.float32),
            pltpu.VMEM((CH, DH), jnp.float32),
            pltpu.VMEM((max(ZCH), DH), jnp.float32),
            pltpu.VMEM((DH,), jnp.float32),
            pltpu.VMEM((DH,), jnp.float32),
            pltpu.VMEM_SHARED((ACC_R, DH), jnp.float32),
            pltpu.SemaphoreType.DMA,
            pltpu.SemaphoreType.DMA,
            pltpu.SemaphoreType.DMA,
            pltpu.SemaphoreType.DMA,
        ],
    )
    return f(h2f, src_r, dst_r, w_r, b, alpha)


@jax.jit
def kernel(x, edge_index, weights, W, b, alpha):
    src = edge_index[0].astype(jnp.int32).reshape(NS, NSG, 1, SG)
    dst = edge_index[1].astype(jnp.int32).reshape(NS, NSG, 1, SG)
    w_r = weights.reshape(NS, NSG, 1, SG)
    W2 = W.reshape(D, NC, DH).transpose(1, 0, 2)
    h2 = _matmul(x, W2)
    h2f = h2.reshape(NC * N_NODES, DH)
    return _sc_aggregate(h2f, src, dst, w_r, b, alpha)
